# Initial kernel scaffold; baseline (speedup 1.0000x reference)
import jax
import jax.numpy as jnp
from jax.experimental import pallas as pl


def kernel(input_ids, values, gene_table, value_table):
    raise NotImplementedError("write your pallas kernel here")


# speedup vs baseline: 1.3487x; 1.3487x over previous
import functools

import jax
import jax.numpy as jnp
from jax import lax
from jax.experimental import pallas as pl
from jax.experimental.pallas import tpu as pltpu
from jax.experimental.pallas import tpu_sc as plsc

_GENE_VOCAB = 60697
_VALUE_VOCAB = 51
_D = 512
_B, _L = 64, 1200
_N = _B * _L
_NC, _NS = 2, 16
_NW = _NC * _NS
_PER_W = _N // _NW
_C = 96
_NCHUNK = _PER_W // _C

_mesh = plsc.VectorSubcoreMesh(core_axis_name="c", subcore_axis_name="s")


@functools.partial(
    pl.kernel,
    mesh=_mesh,
    out_type=jax.ShapeDtypeStruct((_N, _D), jnp.float32),
    scratch_types=[
        pltpu.VMEM((_C,), jnp.int32),
        pltpu.VMEM((_C,), jnp.int32),
        pltpu.VMEM((_C, _D), jnp.float32),
        pltpu.VMEM((_C, _D), jnp.float32),
        pltpu.SemaphoreType.DMA,
        pltpu.SemaphoreType.DMA,
    ],
)
def _sc_embed(ids_hbm, vals_hbm, gene_hbm, vtab_hbm, out_hbm,
              gidx_v, vidx_v, grows_v, vrows_v, sem_g, sem_v):
    wid = lax.axis_index("s") * _NC + lax.axis_index("c")
    base = wid * _PER_W

    def chunk_body(ci, carry):
        off = pl.multiple_of(base + ci * _C, _C)
        pltpu.sync_copy(ids_hbm.at[pl.ds(off, _C)], gidx_v)
        pltpu.sync_copy(vals_hbm.at[pl.ds(off, _C)], vidx_v)
        cp_g = pltpu.async_copy(gene_hbm.at[gidx_v], grows_v, sem_g)
        cp_v = pltpu.async_copy(vtab_hbm.at[vidx_v], vrows_v, sem_v)
        cp_g.wait()
        cp_v.wait()

        def add_row(i, c2):
            for j in range(_D // 16):
                sl = pl.ds(j * 16, 16)
                grows_v[i, sl] = grows_v[i, sl] + vrows_v[i, sl]
            return c2

        lax.fori_loop(0, _C, add_row, 0)
        pltpu.sync_copy(grows_v, out_hbm.at[pl.ds(off, _C)])
        return carry

    lax.fori_loop(0, _NCHUNK, chunk_body, 0)


def kernel(input_ids, values, gene_table, value_table):
    ids = input_ids.reshape(-1).astype(jnp.int32)
    vals = values.reshape(-1).astype(jnp.int32)
    out = _sc_embed(ids, vals, gene_table, value_table)
    return out.reshape(_B, _L, _D)


# speedup vs baseline: 1.3902x; 1.0308x over previous
import functools

import jax
import jax.numpy as jnp
from jax import lax
from jax.experimental import pallas as pl
from jax.experimental.pallas import tpu as pltpu
from jax.experimental.pallas import tpu_sc as plsc

_GENE_VOCAB = 60697
_VALUE_VOCAB = 51
_D = 512
_B, _L = 64, 1200
_N = _B * _L
_NC, _NS = 2, 16
_NW = _NC * _NS
_PER_W = _N // _NW
_C = 48
_NCHUNK = _PER_W // _C
_NK = _NCHUNK // 2

_mesh = plsc.VectorSubcoreMesh(core_axis_name="c", subcore_axis_name="s")


@functools.partial(
    pl.kernel,
    mesh=_mesh,
    out_type=jax.ShapeDtypeStruct((_N, _D), jnp.float32),
    scratch_types=[
        pltpu.VMEM((_PER_W,), jnp.int32),
        pltpu.VMEM((_PER_W,), jnp.int32),
        pltpu.VMEM((_C, _D), jnp.float32),
        pltpu.VMEM((_C, _D), jnp.float32),
        pltpu.VMEM((_C, _D), jnp.float32),
        pltpu.VMEM((_C, _D), jnp.float32),
        pltpu.SemaphoreType.DMA,
        pltpu.SemaphoreType.DMA,
        pltpu.SemaphoreType.DMA,
        pltpu.SemaphoreType.DMA,
        pltpu.SemaphoreType.DMA,
        pltpu.SemaphoreType.DMA,
    ],
)
def _sc_embed(ids_hbm, vals_hbm, gene_hbm, vtab_hbm, out_hbm,
              gidx, vidx, g0, v0, g1, v1, sg0, sv0, sg1, sv1, so0, so1):
    wid = lax.axis_index("s") * _NC + lax.axis_index("c")
    base = wid * _PER_W
    pltpu.sync_copy(ids_hbm.at[pl.ds(base, _PER_W)], gidx)
    pltpu.sync_copy(vals_hbm.at[pl.ds(base, _PER_W)], vidx)

    def issue_gathers(ci, gbuf, vbuf, sg, sv):
        isl = pl.ds(pl.multiple_of(ci * _C, _C), _C)
        pltpu.async_copy(gene_hbm.at[gidx.at[isl]], gbuf, sg)
        pltpu.async_copy(vtab_hbm.at[vidx.at[isl]], vbuf, sv)

    def wait_gathers(ci, gbuf, vbuf, sg, sv):
        isl = pl.ds(pl.multiple_of(ci * _C, _C), _C)
        pltpu.make_async_copy(gene_hbm.at[gidx.at[isl]], gbuf, sg).wait()
        pltpu.make_async_copy(vtab_hbm.at[vidx.at[isl]], vbuf, sv).wait()

    def out_slice(ci):
        return out_hbm.at[pl.ds(pl.multiple_of(base + ci * _C, _C), _C)]

    def add_rows(gbuf, vbuf):
        def body(r, carry):
            for j in range(_D // 16):
                sl = pl.ds(j * 16, 16)
                gbuf[r, sl] = gbuf[r, sl] + vbuf[r, sl]
            return carry
        lax.fori_loop(0, _C, body, 0)

    issue_gathers(0, g0, v0, sg0, sv0)

    def round_body(k, carry):
        a = 2 * k
        b = a + 1

        @pl.when(k > 0)
        def _():
            pltpu.make_async_copy(g1, out_slice(b - 2), so1).wait()

        issue_gathers(b, g1, v1, sg1, sv1)

        wait_gathers(a, g0, v0, sg0, sv0)
        add_rows(g0, v0)
        pltpu.async_copy(g0, out_slice(a), so0)

        @pl.when(k < _NK - 1)
        def _():
            pltpu.make_async_copy(g0, out_slice(a), so0).wait()
            issue_gathers(a + 2, g0, v0, sg0, sv0)

        wait_gathers(b, g1, v1, sg1, sv1)
        add_rows(g1, v1)
        pltpu.async_copy(g1, out_slice(b), so1)
        return carry

    lax.fori_loop(0, _NK, round_body, 0)
    pltpu.make_async_copy(g0, out_slice(_NCHUNK - 2), so0).wait()
    pltpu.make_async_copy(g1, out_slice(_NCHUNK - 1), so1).wait()


def kernel(input_ids, values, gene_table, value_table):
    ids = input_ids.reshape(-1).astype(jnp.int32)
    vals = values.reshape(-1).astype(jnp.int32)
    out = _sc_embed(ids, vals, gene_table, value_table)
    return out.reshape(_B, _L, _D)
